# speedup vs baseline: 1.5438x; 1.5438x over previous
import functools

import jax
import jax.numpy as jnp
from jax import lax
from jax.experimental import pallas as pl
from jax.experimental.pallas import tpu as pltpu
from jax.experimental.pallas import tpu_sc as plsc

N_ROWS = 100000
D = 128
NC = 2
NS = 16
NW = NC * NS

CHUNK = 128
FULL = N_ROWS // CHUNK
REM = N_ROWS - FULL * CHUNK
REM_ROW0 = FULL * CHUNK
BASE_CNT = FULL // NW
EXTRA = FULL % NW
STAGE = (BASE_CNT + 1) * CHUNK

_mesh = plsc.VectorSubcoreMesh(core_axis_name="c", subcore_axis_name="s")


@functools.partial(
    pl.kernel,
    out_type=jax.ShapeDtypeStruct((N_ROWS, D), jnp.float32),
    mesh=_mesh,
    scratch_types=[
        pltpu.VMEM((STAGE,), jnp.int32),
        pltpu.VMEM((CHUNK, D), jnp.float32),
        pltpu.SemaphoreType.DMA,
    ],
)
def _gather_kernel(idx_hbm, tbl_hbm, out_hbm, idx_v, buf, gsem):
    w = lax.axis_index("s") * NC + lax.axis_index("c")
    cnt = BASE_CNT + (w < EXTRA).astype(jnp.int32)
    start = BASE_CNT * w + jnp.minimum(w, EXTRA)
    row0 = start * CHUNK

    pltpu.sync_copy(idx_hbm.at[pl.ds(row0, STAGE)], idx_v)

    def chunk_body(k, carry):
        off = k * CHUNK
        pltpu.async_copy(
            tbl_hbm.at[idx_v.at[pl.ds(off, CHUNK)]], buf, gsem
        ).wait()
        pltpu.sync_copy(buf, out_hbm.at[pl.ds(row0 + off, CHUNK)])
        return carry

    lax.fori_loop(0, cnt, chunk_body, 0)

    @pl.when(w == NW - 1)
    def _():
        off = REM_ROW0 - row0
        pltpu.async_copy(
            tbl_hbm.at[idx_v.at[pl.ds(off, REM)]], buf.at[pl.ds(0, REM)], gsem
        ).wait()
        pltpu.sync_copy(buf.at[pl.ds(0, REM)], out_hbm.at[pl.ds(REM_ROW0, REM)])


def kernel(x, edge_index, batch, emb_table):
    idx = jnp.squeeze(x, axis=1)
    pad_to = (NW - 1) * BASE_CNT * CHUNK + min(NW - 1, EXTRA) * CHUNK + STAGE
    idx = jnp.pad(idx, (0, pad_to - N_ROWS))
    return _gather_kernel(idx, emb_table)
